# Initial kernel scaffold; baseline (speedup 1.0000x reference)
import jax
import jax.numpy as jnp
from jax.experimental import pallas as pl


def kernel(x, edge_index, edge_attr, batch, random_idx, params):
    raise NotImplementedError("write your pallas kernel here")


# speedup vs baseline: 1.7457x; 1.7457x over previous
import functools

import jax
import jax.numpy as jnp
from jax import lax
from jax.experimental import pallas as pl
from jax.experimental.pallas import tpu as pltpu
from jax.experimental.pallas import tpu_sc as plsc

N = 10000
E = 320000
DE = 16
H = 128
G = 128
EPS = 1e-5
BETA = 1e-4

_NCORES = 2
_NSUB = 16
_NW = _NCORES * _NSUB
_CHUNK = 128

E_CPW = 79
E_PAD = _NW * E_CPW * _CHUNK
NACC = 10016
P_CPW = 3
NP_PAD = _NW * P_CPW * _CHUNK
GACC = 144


def _sc_gather(table, idx, cpw):
    vpad = _NW * cpw * _CHUNK
    pw = cpw * _CHUNK
    mesh = plsc.VectorSubcoreMesh(core_axis_name="c", subcore_axis_name="s")

    @functools.partial(
        pl.kernel, mesh=mesh,
        out_type=jax.ShapeDtypeStruct((vpad, H), jnp.float32),
        scratch_types=[pltpu.VMEM((_CHUNK,), jnp.int32),
                       pltpu.VMEM((_CHUNK, H), jnp.float32),
                       pltpu.SemaphoreType.DMA],
    )
    def gk(table_hbm, idx_hbm, out_hbm, idx_v, rows_v, sem):
        wid = lax.axis_index("s") * _NCORES + lax.axis_index("c")

        def body(j, carry):
            base = wid * pw + j * _CHUNK
            pltpu.sync_copy(idx_hbm.at[pl.ds(base, _CHUNK)], idx_v)
            pltpu.async_copy(table_hbm.at[idx_v], rows_v, sem).wait()
            pltpu.sync_copy(rows_v, out_hbm.at[pl.ds(base, _CHUNK)])
            return carry

        lax.fori_loop(0, cpw, body, 0)

    return gk(table, idx)


def _sc_scatter_add(vals, idx, zeros, cpw, tacc):
    pw = cpw * _CHUNK
    mesh = plsc.VectorSubcoreMesh(core_axis_name="c", subcore_axis_name="s")

    @functools.partial(
        pl.kernel, mesh=mesh,
        out_type=jax.ShapeDtypeStruct((2 * tacc, H), jnp.float32),
        scratch_types=[pltpu.VMEM((_CHUNK,), jnp.int32),
                       pltpu.VMEM((_CHUNK, H), jnp.float32),
                       pltpu.VMEM_SHARED((tacc, H), jnp.float32),
                       pltpu.SemaphoreType.DMA],
    )
    def sk(vals_hbm, idx_hbm, zeros_hbm, out_hbm, idx_v, val_v, acc_sh, sem):
        cid = lax.axis_index("c")
        sid = lax.axis_index("s")
        wid = sid * _NCORES + cid

        @pl.when(sid == 0)
        def _():
            pltpu.sync_copy(zeros_hbm, acc_sh)

        plsc.subcore_barrier()

        def body(j, carry):
            base = wid * pw + j * _CHUNK
            pltpu.sync_copy(idx_hbm.at[pl.ds(base, _CHUNK)], idx_v)
            pltpu.sync_copy(vals_hbm.at[pl.ds(base, _CHUNK)], val_v)
            pltpu.sync_copy(val_v, acc_sh.at[idx_v], add=True)
            return carry

        lax.fori_loop(0, cpw, body, 0)
        plsc.subcore_barrier()

        @pl.when(sid == 0)
        def _():
            pltpu.sync_copy(acc_sh, out_hbm.at[pl.ds(cid * tacc, tacc)])

    return sk(vals, idx, zeros)


def _colstats(x, nrows, bs):
    d = x.shape[1]
    nb = nrows // bs

    def kfn(x_ref, s_ref, q_ref):
        @pl.when(pl.program_id(0) == 0)
        def _():
            s_ref[...] = jnp.zeros_like(s_ref)
            q_ref[...] = jnp.zeros_like(q_ref)
        xb = x_ref[...]
        s_ref[...] += jnp.sum(xb, axis=0, keepdims=True)
        q_ref[...] += jnp.sum(xb * xb, axis=0, keepdims=True)

    return pl.pallas_call(
        kfn, grid=(nb,),
        in_specs=[pl.BlockSpec((bs, d), lambda i: (i, 0))],
        out_specs=[pl.BlockSpec((1, d), lambda i: (0, 0)),
                   pl.BlockSpec((1, d), lambda i: (0, 0))],
        out_shape=[jax.ShapeDtypeStruct((1, d), jnp.float32)] * 2,
    )(x)


def _mm(x, w, stats=None, n=None, bs=2048):
    m, d = x.shape
    h = w.shape[1]
    nb = m // bs
    bn = stats is not None

    def kfn(*refs):
        if bn:
            x_ref, w_ref, s_ref, q_ref, o_ref = refs
            mu = s_ref[...] / n
            var = q_ref[...] / n - mu * mu
            xb = (x_ref[...] - mu) * lax.rsqrt(var + EPS) + BETA
        else:
            x_ref, w_ref, o_ref = refs
            xb = x_ref[...]
        o_ref[...] = jnp.dot(xb, w_ref[...], preferred_element_type=jnp.float32)

    ins = [x, w] + ([stats[0], stats[1]] if bn else [])
    in_specs = ([pl.BlockSpec((bs, d), lambda i: (i, 0)),
                 pl.BlockSpec((d, h), lambda i: (0, 0))]
                + ([pl.BlockSpec((1, d), lambda i: (0, 0))] * 2 if bn else []))
    return pl.pallas_call(
        kfn, grid=(nb,), in_specs=in_specs,
        out_specs=pl.BlockSpec((bs, h), lambda i: (i, 0)),
        out_shape=jax.ShapeDtypeStruct((m, h), jnp.float32),
    )(*ins)


def _edgework(hr, hc, e, a, mode):
    bs = 2048
    nb = E_PAD // bs

    def kfn(hr_ref, hc_ref, e_ref, a_ref, *outs):
        i = pl.program_id(0)
        rows = lax.broadcasted_iota(jnp.int32, (bs, 1), 0) + i * bs
        mask = (rows < E).astype(jnp.float32)
        hrb, hcb, eb = hr_ref[...], hc_ref[...], e_ref[...]
        logit = jnp.sum(hrb * a_ref[0:1, :] + hcb * a_ref[1:2, :]
                        + eb * a_ref[2:3, :], axis=1, keepdims=True)
        attn = jax.nn.sigmoid(logit)
        msg = (attn * (hcb + eb)) * mask
        outs[0][...] = msg
        if mode == "main":
            outs[1][...] = jnp.maximum(eb + hrb + hcb, 0.0) * mask
        elif mode == "last":
            edge = jnp.maximum(eb + hrb + hcb, 0.0) * mask
            ec = attn * edge
            eo = (1.0 - attn) * edge
            outs[1][...] = ec
            outs[2][...] = eo

            @pl.when(i == 0)
            def _():
                for r in outs[3:]:
                    r[...] = jnp.zeros_like(r)

            outs[3][...] += jnp.sum(ec, axis=0, keepdims=True)
            outs[4][...] += jnp.sum(ec * ec, axis=0, keepdims=True)
            outs[5][...] += jnp.sum(eo, axis=0, keepdims=True)
            outs[6][...] += jnp.sum(eo * eo, axis=0, keepdims=True)

    big = pl.BlockSpec((bs, H), lambda i: (i, 0))
    acc = pl.BlockSpec((1, H), lambda i: (0, 0))
    big_shape = jax.ShapeDtypeStruct((E_PAD, H), jnp.float32)
    acc_shape = jax.ShapeDtypeStruct((1, H), jnp.float32)
    if mode == "main":
        out_specs, out_shape = [big, big], [big_shape, big_shape]
    elif mode == "last":
        out_specs = [big, big, big, acc, acc, acc, acc]
        out_shape = [big_shape, big_shape, big_shape] + [acc_shape] * 4
    else:
        out_specs, out_shape = [big], [big_shape]

    return pl.pallas_call(
        kfn, grid=(nb,),
        in_specs=[big, big, big, pl.BlockSpec((3, H), lambda i: (0, 0))],
        out_specs=out_specs, out_shape=out_shape,
    )(hr, hc, e, a)


def _combine(h, p0, p1, with_stats, bs=2000):
    nb = N // bs

    def kfn(h_ref, a_ref, b_ref, x_ref, *st):
        xb = jnp.maximum(h_ref[...] + a_ref[...] + b_ref[...], 0.0)
        x_ref[...] = xb
        if with_stats:
            s_ref, q_ref = st

            @pl.when(pl.program_id(0) == 0)
            def _():
                s_ref[...] = jnp.zeros_like(s_ref)
                q_ref[...] = jnp.zeros_like(q_ref)

            s_ref[...] += jnp.sum(xb, axis=0, keepdims=True)
            q_ref[...] += jnp.sum(xb * xb, axis=0, keepdims=True)

    big = pl.BlockSpec((bs, H), lambda i: (i, 0))
    acc = pl.BlockSpec((1, H), lambda i: (0, 0))
    out_specs = [big] + ([acc, acc] if with_stats else [])
    out_shape = ([jax.ShapeDtypeStruct((N, H), jnp.float32)]
                 + ([jax.ShapeDtypeStruct((1, H), jnp.float32)] * 2
                    if with_stats else []))
    out = pl.pallas_call(
        kfn, grid=(nb,),
        in_specs=[big, big, big], out_specs=out_specs, out_shape=out_shape,
    )(h, p0, p1)
    return out if with_stats else out[0]


def _node_attention(x, wna_t, db, bs=2000):
    nb = N // bs

    def kfn(x_ref, w_ref, d_ref, xc_ref, xo_ref, cs, cq, osr, oq):
        xb = x_ref[...]
        l0 = jnp.sum(xb * w_ref[0:1, :], axis=1, keepdims=True)
        l1 = jnp.sum(xb * w_ref[1:2, :], axis=1, keepdims=True)
        p0 = jax.nn.sigmoid(l0 - l1 + d_ref[0, 0])
        xc = p0 * xb
        xo = (1.0 - p0) * xb
        xc_ref[...] = xc
        xo_ref[...] = xo

        @pl.when(pl.program_id(0) == 0)
        def _():
            for r in (cs, cq, osr, oq):
                r[...] = jnp.zeros_like(r)

        cs[...] += jnp.sum(xc, axis=0, keepdims=True)
        cq[...] += jnp.sum(xc * xc, axis=0, keepdims=True)
        osr[...] += jnp.sum(xo, axis=0, keepdims=True)
        oq[...] += jnp.sum(xo * xo, axis=0, keepdims=True)

    big = pl.BlockSpec((bs, H), lambda i: (i, 0))
    acc = pl.BlockSpec((1, H), lambda i: (0, 0))
    return pl.pallas_call(
        kfn, grid=(nb,),
        in_specs=[big,
                  pl.BlockSpec((2, H), lambda i: (0, 0)),
                  pl.BlockSpec((1, 1), lambda i: (0, 0),
                               memory_space=pltpu.SMEM)],
        out_specs=[big, big, acc, acc, acc, acc],
        out_shape=([jax.ShapeDtypeStruct((N, H), jnp.float32)] * 2
                   + [jax.ShapeDtypeStruct((1, H), jnp.float32)] * 4),
    )(x, wna_t, db)


def _readout_all(xc0, xc1, xo0, xo1, ri2, wb):
    def kfn(xc0_ref, xc1_ref, xo0_ref, xo1_ref, ri_ref,
            w1c, b1c, w2c, b2c, w1o, b1o, w2o, b2o, w1x, b1x, w2x, b2x,
            oc_ref, oo_ref, ox_ref):
        xc = xc0_ref[...] + xc1_ref[...]
        xo = xo0_ref[...] + xo1_ref[...]
        col = lax.broadcasted_iota(jnp.int32, (G, G), 1)
        perm = (col == ri_ref[...]).astype(jnp.float32)
        xco = jnp.dot(perm, xc, preferred_element_type=jnp.float32) + xo

        def bn(v):
            mu = jnp.mean(v, axis=0, keepdims=True)
            var = jnp.mean(v * v, axis=0, keepdims=True) - mu * mu
            return (v - mu) * lax.rsqrt(var + EPS) + BETA

        def head(v, w1, b1, w2, b2):
            t = jnp.dot(bn(v), w1[...], preferred_element_type=jnp.float32)
            t = jnp.maximum(t + b1[...], 0.0)
            y = jnp.dot(bn(t), w2[...], preferred_element_type=jnp.float32)
            y = y + b2[...]
            m = jnp.max(y, axis=1, keepdims=True)
            lse = m + jnp.log(jnp.sum(jnp.exp(y - m), axis=1, keepdims=True))
            return y - lse

        oc_ref[...] = head(xc, w1c, b1c, w2c, b2c)
        oo_ref[...] = head(xo, w1o, b1o, w2o, b2o)
        ox_ref[...] = head(xco, w1x, b1x, w2x, b2x)

    ins = [xc0, xc1, xo0, xo1, ri2] + list(wb)
    return pl.pallas_call(
        kfn,
        out_shape=[jax.ShapeDtypeStruct((G, 10), jnp.float32)] * 3,
    )(*ins)


def kernel(x, edge_index, edge_attr, batch, random_idx, params):
    f32 = jnp.float32
    row = edge_index[0]
    col = edge_index[1]
    padlen = E_PAD - E
    row_g = jnp.concatenate([row, jnp.zeros((padlen,), jnp.int32)])
    col_g = jnp.concatenate([col, jnp.zeros((padlen,), jnp.int32)])
    row_s = jnp.concatenate([row, jnp.full((padlen,), N, jnp.int32)])
    ea_pad = jnp.pad(edge_attr, ((0, padlen), (0, 0)))
    zeros_n = jnp.zeros((NACC, H), f32)
    zeros_g = jnp.zeros((GACC, H), f32)
    batch_s = jnp.concatenate(
        [batch.astype(jnp.int32), jnp.full((NP_PAD - N,), G, jnp.int32)])

    def egat_main(xi, stats, ei, p, mode):
        h = _mm(xi, p["Wn"], stats=stats, n=xi.shape[0], bs=2000)
        e = _mm(ei, p["We"], bs=2048)
        hr = _sc_gather(h, row_g, E_CPW)
        hc = _sc_gather(h, col_g, E_CPW)
        outs = _edgework(hr, hc, e, p["a"], mode)
        parts = _sc_scatter_add(outs[0], row_s, zeros_n, E_CPW, NACC)
        p0 = parts[:N]
        p1 = parts[NACC:NACC + N]
        return h, p0, p1, outs

    stats = _colstats(x, N, 2000)
    layer_ps = [params["conv_feat"]] + list(params["convs"])
    e_cur = ea_pad
    x_cur = x
    for li, p in enumerate(layer_ps):
        mode = "last" if li == 3 else "main"
        h, p0, p1, outs = egat_main(x_cur, stats, e_cur, p, mode)
        if mode == "main":
            x_cur, s, q = _combine(h, p0, p1, True)
            stats = (s, q)
            e_cur = outs[1]
        else:
            x_cur = _combine(h, p0, p1, False)
            edge_c, edge_o = outs[1], outs[2]
            ec_stats = (outs[3], outs[4])
            eo_stats = (outs[5], outs[6])

    wna_t = params["Wna"].T
    db = (params["bna"][0] - params["bna"][1]).reshape(1, 1).astype(f32)
    xc, xo, cs, cq, osr, oq = _node_attention(x_cur, wna_t, db)

    def egat_pool(xi, xstats, ei, estats, p):
        h = _mm(xi, p["Wn"], stats=xstats, n=N, bs=2000)
        e = _mm(ei, p["We"], stats=estats, n=E, bs=2048)
        hr = _sc_gather(h, row_g, E_CPW)
        hc = _sc_gather(h, col_g, E_CPW)
        (msg,) = _edgework(hr, hc, e, p["a"], "msgonly")
        parts = _sc_scatter_add(msg, row_s, zeros_n, E_CPW, NACC)
        xout = _combine(h, parts[:N], parts[NACC:NACC + N], False)
        xpad = jnp.pad(xout, ((0, NP_PAD - N), (0, 0)))
        gparts = _sc_scatter_add(xpad, batch_s, zeros_g, P_CPW, GACC)
        return gparts

    gc = egat_pool(xc, (cs, cq), edge_c, ec_stats, params["ctx"])
    go = egat_pool(xo, (osr, oq), edge_o, eo_stats, params["obj"])

    ri2 = random_idx.astype(jnp.int32).reshape(G, 1)
    wb = []
    for tag in ("c", "o", "co"):
        wb += [params["fc1_" + tag + "_W"],
               params["fc1_" + tag + "_b"].reshape(1, H),
               params["fc2_" + tag + "_W"],
               params["fc2_" + tag + "_b"].reshape(1, 10)]
    oc, oo, ox = _readout_all(gc[:G], gc[GACC:GACC + G],
                              go[:G], go[GACC:GACC + G], ri2, wb)
    return jnp.stack([oc, oo, ox])
